# baseline (speedup 1.0000x reference)
import jax
import jax.numpy as jnp
from jax import lax
from jax.experimental import pallas as pl
from jax.experimental.pallas import tpu as pltpu
from jax.experimental.pallas import tpu_sc as plsc

_N = 10000
_E = 320000
_D = 128
_L = 3
_G = 128

_NC = 2
_NS = 16
_NW = _NC * _NS
_EPW = _E // _NW
_CHUNK = 80
_NCHUNK = _EPW // _CHUNK
_RPS = 624
_RTAIL = _N - _NS * _RPS

_BN = 1000
_NBLK = _N // _BN


def _sc_agg_body(h_hbm, src_hbm, dst_hbm, out_hbm, acc, src_all, dst_all,
                 rows0, rows1, rows2, gsem0, gsem1, gsem2,
                 ssem0, ssem1, ssem2):
    c = lax.axis_index("c")
    s = lax.axis_index("s")
    wid = s * _NC + c
    pltpu.sync_copy(src_hbm.at[wid], src_all)
    pltpu.sync_copy(dst_hbm.at[wid], dst_all)

    bufs = (rows0, rows1, rows2)
    gsems = (gsem0, gsem1, gsem2)
    ssems = (ssem0, ssem1, ssem2)

    def gath(j, b):
        return pltpu.async_copy(
            h_hbm.at[src_all.at[pl.ds(j * _CHUNK, _CHUNK)]], bufs[b],
            gsems[b])

    gath(0, 0)
    gath(1, 1)

    row0 = s * _RPS
    pltpu.sync_copy(h_hbm.at[pl.ds(row0, _RPS)], acc.at[pl.ds(row0, _RPS)])

    @pl.when(s == _NS - 1)
    def _():
        pltpu.sync_copy(h_hbm.at[pl.ds(_NS * _RPS, _RTAIL)],
                        acc.at[pl.ds(_NS * _RPS, _RTAIL)])

    plsc.subcore_barrier()

    def proc(r, j, first, prefetch):
        pltpu.make_async_copy(
            h_hbm.at[src_all.at[pl.ds(j * _CHUNK, _CHUNK)]], bufs[r],
            gsems[r]).wait()
        pltpu.async_copy(bufs[r], acc.at[dst_all.at[j]], ssems[r], add=True)
        b2 = (r + 2) % 3
        if prefetch:
            if not first:
                pltpu.make_async_copy(bufs[b2], acc.at[dst_all.at[j]],
                                      ssems[b2]).wait()
            gath(j + 2, b2)

    proc(0, 0, True, True)
    proc(1, 1, False, True)
    proc(2, 2, False, True)

    def step(i, carry):
        proc(0, 3 * i, False, True)
        proc(1, 3 * i + 1, False, True)
        proc(2, 3 * i + 2, False, True)
        return carry

    lax.fori_loop(1, (_NCHUNK - 2) // 3, step, 0)
    proc(0, _NCHUNK - 2, False, False)
    proc(1, _NCHUNK - 1, False, False)
    pltpu.make_async_copy(rows2, acc.at[dst_all.at[_NCHUNK - 3]], ssem2).wait()
    pltpu.make_async_copy(rows0, acc.at[dst_all.at[_NCHUNK - 2]], ssem0).wait()
    pltpu.make_async_copy(rows1, acc.at[dst_all.at[_NCHUNK - 1]], ssem1).wait()
    plsc.subcore_barrier()
    pltpu.sync_copy(acc.at[pl.ds(row0, _RPS)], out_hbm.at[c, pl.ds(row0, _RPS)])

    @pl.when(s == _NS - 1)
    def _():
        pltpu.sync_copy(acc.at[pl.ds(_NS * _RPS, _RTAIL)],
                        out_hbm.at[c, pl.ds(_NS * _RPS, _RTAIL)])


_SC_AGG_CACHE = []


def _sc_agg(h, src, dst):
    if not _SC_AGG_CACHE:
        _SC_AGG_CACHE.append(pl.kernel(
            _sc_agg_body,
            out_type=jax.ShapeDtypeStruct((_NC, _N, _D), jnp.float32),
            mesh=plsc.VectorSubcoreMesh(core_axis_name="c",
                                        subcore_axis_name="s",
                                        num_cores=_NC, num_subcores=_NS),
            compiler_params=pltpu.CompilerParams(use_tc_tiling_on_sc=False),
            scratch_types=[
                pltpu.VMEM_SHARED((_N, _D), jnp.float32),
                pltpu.VMEM((_EPW,), jnp.int32),
                pltpu.VMEM((_NCHUNK, _CHUNK), jnp.int32),
                pltpu.VMEM((_CHUNK, _D), jnp.float32),
                pltpu.VMEM((_CHUNK, _D), jnp.float32),
                pltpu.VMEM((_CHUNK, _D), jnp.float32),
                pltpu.SemaphoreType.DMA,
                pltpu.SemaphoreType.DMA,
                pltpu.SemaphoreType.DMA,
                pltpu.SemaphoreType.DMA,
                pltpu.SemaphoreType.DMA,
                pltpu.SemaphoreType.DMA,
            ],
        ))
    return _SC_AGG_CACHE[0](h, src, dst)


def _fc_body(x_ref, w_ref, o_ref):
    o_ref[...] = lax.dot_general(x_ref[...], w_ref[...],
                                 (((1,), (1,)), ((), ())),
                                 preferred_element_type=jnp.float32)


def _fc(x, w):
    return pl.pallas_call(
        _fc_body,
        grid=(_NBLK,),
        in_specs=[pl.BlockSpec((_BN, _D), lambda i: (i, 0)),
                  pl.BlockSpec((_D, _D), lambda i: (0, 0))],
        out_specs=pl.BlockSpec((_BN, _D), lambda i: (i, 0)),
        out_shape=jax.ShapeDtypeStruct((_N, _D), jnp.float32),
    )(x, w)


def _mlp_body(p_ref, h_ref, w1_ref, b1_ref, w2_ref, b2_ref,
              z_ref, s_ref, q_ref):
    i = pl.program_id(0)
    z0 = p_ref[0] + p_ref[1] - h_ref[...]
    z1 = jnp.maximum(
        lax.dot_general(z0, w1_ref[...], (((1,), (1,)), ((), ())),
                        preferred_element_type=jnp.float32) + b1_ref[...], 0.0)
    z2 = jnp.maximum(
        lax.dot_general(z1, w2_ref[...], (((1,), (1,)), ((), ())),
                        preferred_element_type=jnp.float32) + b2_ref[...], 0.0)
    z_ref[...] = z2
    s = jnp.sum(z2, axis=0, keepdims=True)
    q = jnp.sum(z2 * z2, axis=0, keepdims=True)

    @pl.when(i == 0)
    def _():
        s_ref[...] = s
        q_ref[...] = q

    @pl.when(i > 0)
    def _():
        s_ref[...] += s
        q_ref[...] += q


def _mlp(p, h, w1, b1, w2, b2):
    return pl.pallas_call(
        _mlp_body,
        grid=(_NBLK,),
        in_specs=[pl.BlockSpec((_NC, _BN, _D), lambda i: (0, i, 0)),
                  pl.BlockSpec((_BN, _D), lambda i: (i, 0)),
                  pl.BlockSpec((_D, _D), lambda i: (0, 0)),
                  pl.BlockSpec((1, _D), lambda i: (0, 0)),
                  pl.BlockSpec((_D, _D), lambda i: (0, 0)),
                  pl.BlockSpec((1, _D), lambda i: (0, 0))],
        out_specs=[pl.BlockSpec((_BN, _D), lambda i: (i, 0)),
                   pl.BlockSpec((1, _D), lambda i: (0, 0)),
                   pl.BlockSpec((1, _D), lambda i: (0, 0))],
        out_shape=[jax.ShapeDtypeStruct((_N, _D), jnp.float32),
                   jax.ShapeDtypeStruct((1, _D), jnp.float32),
                   jax.ShapeDtypeStruct((1, _D), jnp.float32)],
    )(p, h, w1, b1, w2, b2)


def _norm_body(z_ref, s_ref, q_ref, g_ref, be_ref, batch_ref, pin_ref,
               h_ref, pout_ref):
    i = pl.program_id(0)
    mean = s_ref[...] / _N
    var = q_ref[...] / _N - mean * mean
    inv = lax.rsqrt(var + 1e-5) * g_ref[...]
    hn = (z_ref[...] - mean) * inv + be_ref[...]
    h_ref[...] = hn
    b = batch_ref[0, 0, :]
    onehot = (b[:, None] == lax.broadcasted_iota(jnp.int32, (_BN, _G), 1)
              ).astype(jnp.float32)
    contrib = lax.dot_general(onehot, hn, (((0,), (0,)), ((), ())),
                              preferred_element_type=jnp.float32)

    @pl.when(i == 0)
    def _():
        pout_ref[...] = pin_ref[...] + contrib

    @pl.when(i > 0)
    def _():
        pout_ref[...] += contrib


def _norm(z, s, q, g, be, batch3, pin):
    return pl.pallas_call(
        _norm_body,
        grid=(_NBLK,),
        in_specs=[pl.BlockSpec((_BN, _D), lambda i: (i, 0)),
                  pl.BlockSpec((1, _D), lambda i: (0, 0)),
                  pl.BlockSpec((1, _D), lambda i: (0, 0)),
                  pl.BlockSpec((1, _D), lambda i: (0, 0)),
                  pl.BlockSpec((1, _D), lambda i: (0, 0)),
                  pl.BlockSpec((1, 1, _BN), lambda i: (i, 0, 0)),
                  pl.BlockSpec((_G, _D), lambda i: (0, 0))],
        out_specs=[pl.BlockSpec((_BN, _D), lambda i: (i, 0)),
                   pl.BlockSpec((_G, _D), lambda i: (0, 0))],
        out_shape=[jax.ShapeDtypeStruct((_N, _D), jnp.float32),
                   jax.ShapeDtypeStruct((_G, _D), jnp.float32)],
    )(z, s, q, g, be, batch3, pin)


def kernel(x, edge_index, batch, fc_W, W1, b1, W2, b2, gamma, beta):
    src = edge_index[0].reshape(_NW, _EPW)
    dst = edge_index[1].reshape(_NW, _NCHUNK, _CHUNK)
    batch3 = batch.reshape(_NBLK, 1, _BN)
    h = _fc(x, fc_W)
    pooled = jnp.zeros((_G, _D), jnp.float32)
    xs = []
    for l in range(_L):
        p = _sc_agg(h, src, dst)
        z, s, q = _mlp(p, h, W1[l], b1[l].reshape(1, _D),
                       W2[l], b2[l].reshape(1, _D))
        h, pooled = _norm(z, s, q, gamma[l].reshape(1, _D),
                          beta[l].reshape(1, _D), batch3, pooled)
        xs.append(h)
    return (pooled, jnp.concatenate(xs, axis=1))


# speedup vs baseline: 1.0302x; 1.0302x over previous
import jax
import jax.numpy as jnp
from jax import lax
from jax.experimental import pallas as pl
from jax.experimental.pallas import tpu as pltpu
from jax.experimental.pallas import tpu_sc as plsc

_N = 10000
_E = 320000
_D = 128
_L = 3
_G = 128

_NC = 2
_NS = 16
_NW = _NC * _NS
_EPW = _E // _NW
_CHUNK = 80
_NCHUNK = _EPW // _CHUNK
_RPS = 624
_RTAIL = _N - _NS * _RPS

_BN = 1000
_NBLK = _N // _BN


def _sc_agg_body(h_hbm, src_hbm, dst_hbm, out_hbm, acc, src_all, dst_all,
                 rows0, rows1, rows2, gsem0, gsem1, gsem2,
                 ssem0, ssem1, ssem2):
    c = lax.axis_index("c")
    s = lax.axis_index("s")
    wid = s * _NC + c
    pltpu.sync_copy(src_hbm.at[wid], src_all)
    pltpu.sync_copy(dst_hbm.at[wid], dst_all)

    bufs = (rows0, rows1, rows2)
    gsems = (gsem0, gsem1, gsem2)
    ssems = (ssem0, ssem1, ssem2)

    def gath(j, b):
        return pltpu.async_copy(
            h_hbm.at[src_all.at[pl.ds(j * _CHUNK, _CHUNK)]], bufs[b],
            gsems[b])

    gath(0, 0)
    gath(1, 1)

    row0 = s * _RPS
    pltpu.sync_copy(h_hbm.at[pl.ds(row0, _RPS)], acc.at[pl.ds(row0, _RPS)])

    @pl.when(s == _NS - 1)
    def _():
        pltpu.sync_copy(h_hbm.at[pl.ds(_NS * _RPS, _RTAIL)],
                        acc.at[pl.ds(_NS * _RPS, _RTAIL)])

    plsc.subcore_barrier()

    def proc(r, j, first=False, prefetch=True):
        b2 = (r + 2) % 3
        if prefetch:
            if not first:
                pltpu.make_async_copy(bufs[b2], acc.at[dst_all.at[j]],
                                      ssems[b2]).wait()
            gath(j + 2, b2)
        pltpu.make_async_copy(
            h_hbm.at[src_all.at[pl.ds(j * _CHUNK, _CHUNK)]], bufs[r],
            gsems[r]).wait()
        pltpu.async_copy(bufs[r], acc.at[dst_all.at[j]], ssems[r], add=True)

    proc(0, 0, first=True)
    proc(1, 1)
    proc(2, 2)

    def step(i, carry):
        proc(0, 3 * i)
        proc(1, 3 * i + 1)
        proc(2, 3 * i + 2)
        return carry

    lax.fori_loop(1, (_NCHUNK - 2) // 3, step, 0)
    proc(0, _NCHUNK - 2, prefetch=False)
    proc(1, _NCHUNK - 1, prefetch=False)
    pltpu.make_async_copy(rows2, acc.at[dst_all.at[_NCHUNK - 3]], ssem2).wait()
    pltpu.make_async_copy(rows0, acc.at[dst_all.at[_NCHUNK - 2]], ssem0).wait()
    pltpu.make_async_copy(rows1, acc.at[dst_all.at[_NCHUNK - 1]], ssem1).wait()
    plsc.subcore_barrier()
    pltpu.sync_copy(acc.at[pl.ds(row0, _RPS)], out_hbm.at[c, pl.ds(row0, _RPS)])

    @pl.when(s == _NS - 1)
    def _():
        pltpu.sync_copy(acc.at[pl.ds(_NS * _RPS, _RTAIL)],
                        out_hbm.at[c, pl.ds(_NS * _RPS, _RTAIL)])


_SC_AGG_CACHE = []


def _sc_agg(h, src, dst):
    if not _SC_AGG_CACHE:
        _SC_AGG_CACHE.append(pl.kernel(
            _sc_agg_body,
            out_type=jax.ShapeDtypeStruct((_NC, _N, _D), jnp.float32),
            mesh=plsc.VectorSubcoreMesh(core_axis_name="c",
                                        subcore_axis_name="s",
                                        num_cores=_NC, num_subcores=_NS),
            compiler_params=pltpu.CompilerParams(use_tc_tiling_on_sc=False),
            scratch_types=[
                pltpu.VMEM_SHARED((_N, _D), jnp.float32),
                pltpu.VMEM((_EPW,), jnp.int32),
                pltpu.VMEM((_NCHUNK, _CHUNK), jnp.int32),
                pltpu.VMEM((_CHUNK, _D), jnp.float32),
                pltpu.VMEM((_CHUNK, _D), jnp.float32),
                pltpu.VMEM((_CHUNK, _D), jnp.float32),
                pltpu.SemaphoreType.DMA,
                pltpu.SemaphoreType.DMA,
                pltpu.SemaphoreType.DMA,
                pltpu.SemaphoreType.DMA,
                pltpu.SemaphoreType.DMA,
                pltpu.SemaphoreType.DMA,
            ],
        ))
    return _SC_AGG_CACHE[0](h, src, dst)


def _fc_body(x_ref, w_ref, o_ref):
    o_ref[...] = lax.dot_general(x_ref[...], w_ref[...],
                                 (((1,), (1,)), ((), ())),
                                 preferred_element_type=jnp.float32)


def _fc(x, w):
    return pl.pallas_call(
        _fc_body,
        grid=(_NBLK,),
        in_specs=[pl.BlockSpec((_BN, _D), lambda i: (i, 0)),
                  pl.BlockSpec((_D, _D), lambda i: (0, 0))],
        out_specs=pl.BlockSpec((_BN, _D), lambda i: (i, 0)),
        out_shape=jax.ShapeDtypeStruct((_N, _D), jnp.float32),
    )(x, w)


def _mlp_body(p_ref, h_ref, w1_ref, b1_ref, w2_ref, b2_ref,
              z_ref, s_ref, q_ref):
    i = pl.program_id(0)
    z0 = p_ref[0] + p_ref[1] - h_ref[...]
    z1 = jnp.maximum(
        lax.dot_general(z0, w1_ref[...], (((1,), (1,)), ((), ())),
                        preferred_element_type=jnp.float32) + b1_ref[...], 0.0)
    z2 = jnp.maximum(
        lax.dot_general(z1, w2_ref[...], (((1,), (1,)), ((), ())),
                        preferred_element_type=jnp.float32) + b2_ref[...], 0.0)
    z_ref[...] = z2
    s = jnp.sum(z2, axis=0, keepdims=True)
    q = jnp.sum(z2 * z2, axis=0, keepdims=True)

    @pl.when(i == 0)
    def _():
        s_ref[...] = s
        q_ref[...] = q

    @pl.when(i > 0)
    def _():
        s_ref[...] += s
        q_ref[...] += q


def _mlp(p, h, w1, b1, w2, b2):
    return pl.pallas_call(
        _mlp_body,
        grid=(_NBLK,),
        in_specs=[pl.BlockSpec((_NC, _BN, _D), lambda i: (0, i, 0)),
                  pl.BlockSpec((_BN, _D), lambda i: (i, 0)),
                  pl.BlockSpec((_D, _D), lambda i: (0, 0)),
                  pl.BlockSpec((1, _D), lambda i: (0, 0)),
                  pl.BlockSpec((_D, _D), lambda i: (0, 0)),
                  pl.BlockSpec((1, _D), lambda i: (0, 0))],
        out_specs=[pl.BlockSpec((_BN, _D), lambda i: (i, 0)),
                   pl.BlockSpec((1, _D), lambda i: (0, 0)),
                   pl.BlockSpec((1, _D), lambda i: (0, 0))],
        out_shape=[jax.ShapeDtypeStruct((_N, _D), jnp.float32),
                   jax.ShapeDtypeStruct((1, _D), jnp.float32),
                   jax.ShapeDtypeStruct((1, _D), jnp.float32)],
    )(p, h, w1, b1, w2, b2)


def _norm_body(z_ref, s_ref, q_ref, g_ref, be_ref, batch_ref, pin_ref,
               h_ref, pout_ref):
    i = pl.program_id(0)
    mean = s_ref[...] / _N
    var = q_ref[...] / _N - mean * mean
    inv = lax.rsqrt(var + 1e-5) * g_ref[...]
    hn = (z_ref[...] - mean) * inv + be_ref[...]
    h_ref[...] = hn
    b = batch_ref[0, 0, :]
    onehot = (b[:, None] == lax.broadcasted_iota(jnp.int32, (_BN, _G), 1)
              ).astype(jnp.float32)
    contrib = lax.dot_general(onehot, hn, (((0,), (0,)), ((), ())),
                              preferred_element_type=jnp.float32)

    @pl.when(i == 0)
    def _():
        pout_ref[...] = pin_ref[...] + contrib

    @pl.when(i > 0)
    def _():
        pout_ref[...] += contrib


def _norm(z, s, q, g, be, batch3, pin):
    return pl.pallas_call(
        _norm_body,
        grid=(_NBLK,),
        in_specs=[pl.BlockSpec((_BN, _D), lambda i: (i, 0)),
                  pl.BlockSpec((1, _D), lambda i: (0, 0)),
                  pl.BlockSpec((1, _D), lambda i: (0, 0)),
                  pl.BlockSpec((1, _D), lambda i: (0, 0)),
                  pl.BlockSpec((1, _D), lambda i: (0, 0)),
                  pl.BlockSpec((1, 1, _BN), lambda i: (i, 0, 0)),
                  pl.BlockSpec((_G, _D), lambda i: (0, 0))],
        out_specs=[pl.BlockSpec((_BN, _D), lambda i: (i, 0)),
                   pl.BlockSpec((_G, _D), lambda i: (0, 0))],
        out_shape=[jax.ShapeDtypeStruct((_N, _D), jnp.float32),
                   jax.ShapeDtypeStruct((_G, _D), jnp.float32)],
    )(z, s, q, g, be, batch3, pin)


def kernel(x, edge_index, batch, fc_W, W1, b1, W2, b2, gamma, beta):
    src = edge_index[0].reshape(_NW, _EPW)
    dst = edge_index[1].reshape(_NW, _NCHUNK, _CHUNK)
    batch3 = batch.reshape(_NBLK, 1, _BN)
    h = _fc(x, fc_W)
    pooled = jnp.zeros((_G, _D), jnp.float32)
    xs = []
    for l in range(_L):
        p = _sc_agg(h, src, dst)
        z, s, q = _mlp(p, h, W1[l], b1[l].reshape(1, _D),
                       W2[l], b2[l].reshape(1, _D))
        h, pooled = _norm(z, s, q, gamma[l].reshape(1, _D),
                          beta[l].reshape(1, _D), batch3, pooled)
        xs.append(h)
    return (pooled, jnp.concatenate(xs, axis=1))


# speedup vs baseline: 1.0630x; 1.0318x over previous
import jax
import jax.numpy as jnp
from jax import lax
from jax.experimental import pallas as pl
from jax.experimental.pallas import tpu as pltpu
from jax.experimental.pallas import tpu_sc as plsc

_N = 10000
_E = 320000
_D = 128
_L = 3
_G = 128

_NC = 2
_NS = 16
_NW = _NC * _NS
_EPW = _E // _NW
_CHUNK = 80
_NCHUNK = _EPW // _CHUNK
_RPS = 624
_RTAIL = _N - _NS * _RPS

_BN = 1000
_NBLK = _N // _BN


def _sc_agg_body(h_hbm, src_hbm, dst_hbm, out_hbm, acc, src_all, dst_all,
                 rows0, rows1, rows2, gsem0, gsem1, gsem2,
                 ssem0, ssem1, ssem2):
    c = lax.axis_index("c")
    s = lax.axis_index("s")
    wid = s * _NC + c
    pltpu.sync_copy(src_hbm.at[wid], src_all)
    pltpu.sync_copy(dst_hbm.at[wid], dst_all)

    bufs = (rows0, rows1, rows2)
    gsems = (gsem0, gsem1, gsem2)
    ssems = (ssem0, ssem1, ssem2)

    def gath(j, b):
        return pltpu.async_copy(
            h_hbm.at[src_all.at[pl.ds(j * _CHUNK, _CHUNK)]], bufs[b],
            gsems[b])

    gath(0, 0)
    gath(1, 1)

    row0 = s * _RPS
    pltpu.sync_copy(h_hbm.at[pl.ds(row0, _RPS)], acc.at[pl.ds(row0, _RPS)])

    @pl.when(s == _NS - 1)
    def _():
        pltpu.sync_copy(h_hbm.at[pl.ds(_NS * _RPS, _RTAIL)],
                        acc.at[pl.ds(_NS * _RPS, _RTAIL)])

    plsc.subcore_barrier()

    def proc(r, j, first=False, prefetch=True):
        b2 = (r + 2) % 3
        if prefetch:
            if not first:
                pltpu.make_async_copy(bufs[b2], acc.at[dst_all.at[j]],
                                      ssems[b2]).wait()
            gath(j + 2, b2)
        pltpu.make_async_copy(
            h_hbm.at[src_all.at[pl.ds(j * _CHUNK, _CHUNK)]], bufs[r],
            gsems[r]).wait()
        pltpu.async_copy(bufs[r], acc.at[dst_all.at[j]], ssems[r], add=True)

    proc(0, 0, first=True)
    proc(1, 1)
    proc(2, 2)

    def step(i, carry):
        proc(0, 3 * i)
        proc(1, 3 * i + 1)
        proc(2, 3 * i + 2)
        return carry

    lax.fori_loop(1, (_NCHUNK - 2) // 3, step, 0)
    proc(0, _NCHUNK - 2, prefetch=False)
    proc(1, _NCHUNK - 1, prefetch=False)
    pltpu.make_async_copy(rows2, acc.at[dst_all.at[_NCHUNK - 3]], ssem2).wait()
    pltpu.make_async_copy(rows0, acc.at[dst_all.at[_NCHUNK - 2]], ssem0).wait()
    pltpu.make_async_copy(rows1, acc.at[dst_all.at[_NCHUNK - 1]], ssem1).wait()
    plsc.subcore_barrier()
    pltpu.sync_copy(acc.at[pl.ds(row0, _RPS)], out_hbm.at[c, pl.ds(row0, _RPS)])

    @pl.when(s == _NS - 1)
    def _():
        pltpu.sync_copy(acc.at[pl.ds(_NS * _RPS, _RTAIL)],
                        out_hbm.at[c, pl.ds(_NS * _RPS, _RTAIL)])


_SC_AGG_CACHE = []


def _sc_agg(h, src, dst):
    if not _SC_AGG_CACHE:
        _SC_AGG_CACHE.append(pl.kernel(
            _sc_agg_body,
            out_type=jax.ShapeDtypeStruct((_NC, _N, _D), jnp.float32),
            mesh=plsc.VectorSubcoreMesh(core_axis_name="c",
                                        subcore_axis_name="s",
                                        num_cores=_NC, num_subcores=_NS),
            compiler_params=pltpu.CompilerParams(use_tc_tiling_on_sc=False),
            scratch_types=[
                pltpu.VMEM_SHARED((_N, _D), jnp.float32),
                pltpu.VMEM((_EPW,), jnp.int32),
                pltpu.VMEM((_NCHUNK, _CHUNK), jnp.int32),
                pltpu.VMEM((_CHUNK, _D), jnp.float32),
                pltpu.VMEM((_CHUNK, _D), jnp.float32),
                pltpu.VMEM((_CHUNK, _D), jnp.float32),
                pltpu.SemaphoreType.DMA,
                pltpu.SemaphoreType.DMA,
                pltpu.SemaphoreType.DMA,
                pltpu.SemaphoreType.DMA,
                pltpu.SemaphoreType.DMA,
                pltpu.SemaphoreType.DMA,
            ],
        ))
    return _SC_AGG_CACHE[0](h, src, dst)


def _fc_body(x_ref, w_ref, o_ref):
    o_ref[...] = lax.dot_general(x_ref[...], w_ref[...],
                                 (((1,), (1,)), ((), ())),
                                 preferred_element_type=jnp.float32)


def _fc(x, w):
    return pl.pallas_call(
        _fc_body,
        grid=(_NBLK,),
        in_specs=[pl.BlockSpec((_BN, _D), lambda i: (i, 0)),
                  pl.BlockSpec((_D, _D), lambda i: (0, 0))],
        out_specs=pl.BlockSpec((_BN, _D), lambda i: (i, 0)),
        out_shape=jax.ShapeDtypeStruct((_N, _D), jnp.float32),
    )(x, w)


def _layer_body(p_ref, h_ref, w1_ref, b1_ref, w2_ref, b2_ref, g_ref, be_ref,
                batch_ref, pin_ref, h_out_ref, pout_ref, z_scr, s_scr, q_scr):
    ph = pl.program_id(0)
    i = pl.program_id(1)

    @pl.when(ph == 0)
    def _():
        z0 = p_ref[0] + p_ref[1] - h_ref[...]
        z1 = jnp.maximum(
            lax.dot_general(z0, w1_ref[...], (((1,), (1,)), ((), ())),
                            preferred_element_type=jnp.float32)
            + b1_ref[...], 0.0)
        z2 = jnp.maximum(
            lax.dot_general(z1, w2_ref[...], (((1,), (1,)), ((), ())),
                            preferred_element_type=jnp.float32)
            + b2_ref[...], 0.0)
        z_scr[pl.ds(i * _BN, _BN), :] = z2
        s = jnp.sum(z2, axis=0, keepdims=True)
        q = jnp.sum(z2 * z2, axis=0, keepdims=True)

        @pl.when(i == 0)
        def _():
            s_scr[...] = s
            q_scr[...] = q

        @pl.when(i > 0)
        def _():
            s_scr[...] += s
            q_scr[...] += q

    @pl.when(ph == 1)
    def _():
        mean = s_scr[...] / _N
        var = q_scr[...] / _N - mean * mean
        inv = lax.rsqrt(var + 1e-5) * g_ref[...]
        hn = (z_scr[pl.ds(i * _BN, _BN), :] - mean) * inv + be_ref[...]
        h_out_ref[...] = hn
        b = batch_ref[0, 0, :]
        onehot = (b[:, None] == lax.broadcasted_iota(jnp.int32, (_BN, _G), 1)
                  ).astype(jnp.float32)
        contrib = lax.dot_general(onehot, hn, (((0,), (0,)), ((), ())),
                                  preferred_element_type=jnp.float32)

        @pl.when(i == 0)
        def _():
            pout_ref[...] = pin_ref[...] + contrib

        @pl.when(i > 0)
        def _():
            pout_ref[...] += contrib


def _layer(p, h, w1, b1, w2, b2, g, be, batch3, pin):
    return pl.pallas_call(
        _layer_body,
        grid=(2, _NBLK),
        in_specs=[
            pl.BlockSpec((_NC, _BN, _D), lambda ph, i: (0, i * (1 - ph), 0)),
            pl.BlockSpec((_BN, _D), lambda ph, i: (i * (1 - ph), 0)),
            pl.BlockSpec((_D, _D), lambda ph, i: (0, 0)),
            pl.BlockSpec((1, _D), lambda ph, i: (0, 0)),
            pl.BlockSpec((_D, _D), lambda ph, i: (0, 0)),
            pl.BlockSpec((1, _D), lambda ph, i: (0, 0)),
            pl.BlockSpec((1, _D), lambda ph, i: (0, 0)),
            pl.BlockSpec((1, _D), lambda ph, i: (0, 0)),
            pl.BlockSpec((1, 1, _BN), lambda ph, i: (i * ph, 0, 0)),
            pl.BlockSpec((_G, _D), lambda ph, i: (0, 0)),
        ],
        out_specs=[pl.BlockSpec((_BN, _D), lambda ph, i: (i * ph, 0)),
                   pl.BlockSpec((_G, _D), lambda ph, i: (0, 0))],
        out_shape=[jax.ShapeDtypeStruct((_N, _D), jnp.float32),
                   jax.ShapeDtypeStruct((_G, _D), jnp.float32)],
        scratch_shapes=[pltpu.VMEM((_N, _D), jnp.float32),
                        pltpu.VMEM((1, _D), jnp.float32),
                        pltpu.VMEM((1, _D), jnp.float32)],
    )(p, h, w1, b1, w2, b2, g, be, batch3, pin)


def kernel(x, edge_index, batch, fc_W, W1, b1, W2, b2, gamma, beta):
    src = edge_index[0].reshape(_NW, _EPW)
    dst = edge_index[1].reshape(_NW, _NCHUNK, _CHUNK)
    batch3 = batch.reshape(_NBLK, 1, _BN)
    h = _fc(x, fc_W)
    pooled = jnp.zeros((_G, _D), jnp.float32)
    xs = []
    for l in range(_L):
        p = _sc_agg(h, src, dst)
        h, pooled = _layer(p, h, W1[l], b1[l].reshape(1, _D),
                           W2[l], b2[l].reshape(1, _D),
                           gamma[l].reshape(1, _D), beta[l].reshape(1, _D),
                           batch3, pooled)
        xs.append(h)
    return (pooled, jnp.concatenate(xs, axis=1))
